# baseline (speedup 1.0000x reference)
import jax
import jax.numpy as jnp
from jax import lax
from jax.experimental import pallas as pl
from jax.experimental.pallas import tpu as pltpu
from jax.experimental.pallas import tpu_sc as plsc

B = 4096
N = 100
D = 64
BUCKETS = 256
ROWS = BUCKETS + 1
NW = 32
LANES = 16
QB = 1024
NQ = B // QB
NU = N * NQ
PAN_B = 128
PPU = QB // PAN_B


def _prep_body(alphaT_ref, maskT_ref, basis_ref, missing_ref, scale_ref,
               bias_ref, me_ref, ve3_ref, sm_ref, spw_ref, cat_ref,
               aT_ref, gT_ref, t2_ref):
    sem = lax.dot_general(sm_ref[...], spw_ref[...], (((1,), (1,)), ((), ())),
                          preferred_element_type=jnp.float32)
    me = me_ref[...]
    c1 = sem + me[1:2, :]
    c0 = sem + me[0:1, :] + missing_ref[...]
    catf = cat_ref[...]
    t2_ref[:, 0:BUCKETS, :] = (ve3_ref[...] * catf[:, None, None]
                               + c1[:, None, :])
    t2_ref[:, BUCKETS:ROWS, :] = c0[:, None, :]

    alphaT = alphaT_ref[...]
    maskT = maskT_ref[...]
    mask_f = maskT.astype(jnp.float32)
    aT_ref[...] = mask_f * (alphaT * scale_ref[...][:, None]
                            + bias_ref[...][:, None])
    bucket = jnp.clip(
        jnp.round((jnp.clip(alphaT, -1.0, 1.0) + 1.0) * 0.5 * (BUCKETS - 1)),
        0, BUCKETS - 1).astype(jnp.int32)
    gT_ref[...] = jnp.where(maskT == 1, bucket, BUCKETS)


def _sc_body(aT_hbm, gT_hbm, basis_hbm, t2_hbm, cat_hbm, out_hbm,
             slab0, slab1, a0, a1, g0, g1, basis_v, cat_v, pan0, pan1,
             sem_in, sem_o):
    wid = lax.axis_index("s") * 2 + lax.axis_index("c")
    pltpu.sync_copy(basis_hbm, basis_v)
    pltpu.sync_copy(cat_hbm, cat_v)

    slabs = (slab0, slab1)
    avs = (a0, a1)
    gvs = (g0, g1)
    pans = (pan0, pan1)
    nunits = jnp.where(wid < NU % NW, NU // NW + 1, NU // NW)

    def in_descs(u, p):
        n = u >> 2
        q = u & 3
        return (
            pltpu.make_async_copy(t2_hbm.at[n], slabs[p], sem_in),
            pltpu.make_async_copy(aT_hbm.at[n, pl.ds(q * QB, QB)],
                                  avs[p], sem_in),
            pltpu.make_async_copy(gT_hbm.at[n, pl.ds(q * QB, QB)],
                                  gvs[p], sem_in),
        )

    def out_desc(u, pp, pan):
        n = u >> 2
        bt = (u & 3) * PPU + pp
        return pltpu.make_async_copy(pan, out_hbm.at[n, :, bt], sem_o)

    def compute(n, pp, slab, a_v, g_v, pan):
        base = pp * PAN_B
        av = [a_v[pl.ds(base + LANES * l, LANES)]
              for l in range(PAN_B // LANES)]
        gv = [g_v[pl.ds(base + LANES * l, LANES)]
              for l in range(PAN_B // LANES)]

        @plsc.parallel_loop(0, D, unroll=8)
        def _(d):
            bs = plsc.load_gather(
                basis_v, [jnp.full((LANES,), n * D + d, jnp.int32)])
            dfull = jnp.full((LANES,), d, jnp.int32)
            dt = lax.shift_right_logical(d, 3)
            dr = lax.bitwise_and(d, 7)
            for l in range(PAN_B // LANES):
                v = plsc.load_gather(slab, [gv[l], dfull])
                pan[dt, dr, pl.ds(LANES * l, LANES)] = av[l] * bs + v

    def compute_const(n, pp, slab, a_v, g_v, pan):
        base = pp * PAN_B
        av = [a_v[pl.ds(base + LANES * l, LANES)]
              for l in range(PAN_B // LANES)]
        mv = [g_v[pl.ds(base + LANES * l, LANES)] == BUCKETS
              for l in range(PAN_B // LANES)]

        @plsc.parallel_loop(0, D, unroll=8)
        def _(d):
            dfull = jnp.full((LANES,), d, jnp.int32)
            bs = plsc.load_gather(
                basis_v, [jnp.full((LANES,), n * D + d, jnp.int32)])
            c1s = plsc.load_gather(slab, [jnp.zeros((LANES,), jnp.int32),
                                          dfull])
            c0s = plsc.load_gather(slab, [jnp.full((LANES,), BUCKETS,
                                                   jnp.int32), dfull])
            dt = lax.shift_right_logical(d, 3)
            dr = lax.bitwise_and(d, 7)
            for l in range(PAN_B // LANES):
                v = jnp.where(mv[l], c0s, c1s)
                pan[dt, dr, pl.ds(LANES * l, LANES)] = av[l] * bs + v

    for dsc in in_descs(wid, 0):
        dsc.start()

    def unit_body(t, _):
        u = wid + NW * t
        p = t % 2

        for phase in range(2):
            @pl.when(p == phase)
            def _():
                slab, a_v, g_v = slabs[phase], avs[phase], gvs[phase]

                @pl.when(t + 1 < nunits)
                def _():
                    for dsc in in_descs(u + NW, 1 - phase):
                        dsc.start()

                for dsc in in_descs(u, phase):
                    dsc.wait()

                cs = jnp.max(plsc.load_gather(
                    cat_v, [jnp.full((LANES,), u >> 2, jnp.int32)]))

                def panel_body(pp, _):
                    for pphase in range(2):
                        @pl.when(pp % 2 == pphase)
                        def _():
                            pan = pans[pphase]

                            @pl.when(t * PPU + pp >= 2)
                            def _():
                                out_desc(u, pp, pan).wait()

                            @pl.when(cs > 0.0)
                            def _():
                                compute(u >> 2, pp, slab, a_v, g_v, pan)

                            @pl.when(cs == 0.0)
                            def _():
                                compute_const(u >> 2, pp, slab, a_v, g_v,
                                              pan)

                            out_desc(u, pp, pan).start()
                    return 0
                lax.fori_loop(0, PPU, panel_body, 0)
        return 0

    lax.fori_loop(0, nunits, unit_body, 0)
    out_desc(0, 0, pans[0]).wait()
    out_desc(0, 0, pans[1]).wait()


def kernel(alpha, mask, basis, missing_basis, alpha_scale, alpha_bias,
           mask_embedding, value_embedding, semantic_matrix, semantic_proj_w,
           categorical_value_mask):
    ve3 = value_embedding.reshape(N, BUCKETS, D)
    catf = categorical_value_mask.astype(jnp.float32)
    aT, gT, t2 = pl.pallas_call(
        _prep_body,
        out_shape=(
            jax.ShapeDtypeStruct((N, B), jnp.float32),
            jax.ShapeDtypeStruct((N, B), jnp.int32),
            jax.ShapeDtypeStruct((N, ROWS, D), jnp.float32),
        ),
    )(alpha.T, mask.T, basis, missing_basis, alpha_scale, alpha_bias,
      mask_embedding, ve3, semantic_matrix, semantic_proj_w, catf)

    mesh = plsc.VectorSubcoreMesh(core_axis_name="c", subcore_axis_name="s")
    sc = pl.kernel(
        _sc_body, mesh=mesh,
        compiler_params=pltpu.CompilerParams(needs_layout_passes=False,
                                             use_tc_tiling_on_sc=False),
        out_type=jax.ShapeDtypeStruct((N, D // 8, B // PAN_B, 8, PAN_B),
                                      jnp.float32),
        scratch_types=[
            pltpu.VMEM((ROWS, D), jnp.float32),
            pltpu.VMEM((ROWS, D), jnp.float32),
            pltpu.VMEM((QB,), jnp.float32),
            pltpu.VMEM((QB,), jnp.float32),
            pltpu.VMEM((QB,), jnp.int32),
            pltpu.VMEM((QB,), jnp.int32),
            pltpu.VMEM((N * D,), jnp.float32),
            pltpu.VMEM((104,), jnp.float32),
            pltpu.VMEM((D // 8, 8, PAN_B), jnp.float32),
            pltpu.VMEM((D // 8, 8, PAN_B), jnp.float32),
            pltpu.SemaphoreType.DMA,
            pltpu.SemaphoreType.DMA,
        ],
    )
    out5 = sc(aT, gT, basis.reshape(N * D), t2, jnp.pad(catf, (0, 4)))
    return out5.transpose(2, 4, 0, 1, 3).reshape(B, N, D)


# speedup vs baseline: 1.1632x; 1.1632x over previous
import jax
import jax.numpy as jnp
from jax import lax
from jax.experimental import pallas as pl
from jax.experimental.pallas import tpu as pltpu
from jax.experimental.pallas import tpu_sc as plsc

B = 4096
N = 100
D = 64
BUCKETS = 256
ROWS = BUCKETS + 1
NW = 32
LANES = 16
QB = 1024
NQ = B // QB
NU = N * NQ
PAN_B = 128
PPU = QB // PAN_B


def _prep_body(alphaT_ref, maskT_ref, basis_ref, missing_ref, scale_ref,
               bias_ref, me_ref, ve3_ref, sm_ref, spw_ref, cat_ref,
               aT_ref, gT_ref, t2_ref):
    sem = lax.dot_general(sm_ref[...], spw_ref[...], (((1,), (1,)), ((), ())),
                          preferred_element_type=jnp.float32)
    me = me_ref[...]
    c1 = sem + me[1:2, :]
    c0 = sem + me[0:1, :] + missing_ref[...]
    catf = cat_ref[...]
    t2_ref[:, 0:BUCKETS, :] = (ve3_ref[...] * catf[:, None, None]
                               + c1[:, None, :])
    t2_ref[:, BUCKETS:ROWS, :] = c0[:, None, :]

    alphaT = alphaT_ref[...]
    maskT = maskT_ref[...]
    mask_f = maskT.astype(jnp.float32)
    aT_ref[...] = mask_f * (alphaT * scale_ref[...][:, None]
                            + bias_ref[...][:, None])
    bucket = jnp.clip(
        jnp.round((jnp.clip(alphaT, -1.0, 1.0) + 1.0) * 0.5 * (BUCKETS - 1)),
        0, BUCKETS - 1).astype(jnp.int32)
    gT_ref[...] = jnp.where(maskT == 1, bucket, BUCKETS)


def _sc_body(aT_hbm, gT_hbm, basis_hbm, t2_hbm, out_hbm,
             slab0, slab1, a0, a1, g0, g1, basis_v, pan0, pan1,
             sem_in, sem_o):
    wid = lax.axis_index("s") * 2 + lax.axis_index("c")
    pltpu.sync_copy(basis_hbm, basis_v)

    slabs = (slab0, slab1)
    avs = (a0, a1)
    gvs = (g0, g1)
    pans = (pan0, pan1)
    nunits = jnp.where(wid < NU % NW, NU // NW + 1, NU // NW)

    def in_descs(u, p):
        n = u >> 2
        q = u & 3
        return (
            pltpu.make_async_copy(t2_hbm.at[n], slabs[p], sem_in),
            pltpu.make_async_copy(aT_hbm.at[n, pl.ds(q * QB, QB)],
                                  avs[p], sem_in),
            pltpu.make_async_copy(gT_hbm.at[n, pl.ds(q * QB, QB)],
                                  gvs[p], sem_in),
        )

    def out_desc(u, pp, pan):
        n = u >> 2
        bt = (u & 3) * PPU + pp
        return pltpu.make_async_copy(pan, out_hbm.at[n, :, bt], sem_o)

    def compute(n, pp, slab, a_v, g_v, pan):
        base = pp * PAN_B
        av = [a_v[pl.ds(base + LANES * l, LANES)]
              for l in range(PAN_B // LANES)]
        gv = [g_v[pl.ds(base + LANES * l, LANES)]
              for l in range(PAN_B // LANES)]

        @plsc.parallel_loop(0, D, unroll=2)
        def _(d):
            bs = plsc.load_gather(
                basis_v, [jnp.full((LANES,), n * D + d, jnp.int32)])
            dfull = jnp.full((LANES,), d, jnp.int32)
            dt = lax.shift_right_logical(d, 3)
            dr = lax.bitwise_and(d, 7)
            for l in range(PAN_B // LANES):
                v = plsc.load_gather(slab, [gv[l], dfull])
                pan[dt, dr, pl.ds(LANES * l, LANES)] = av[l] * bs + v

    for dsc in in_descs(wid, 0):
        dsc.start()

    def unit_body(t, _):
        u = wid + NW * t
        p = t % 2

        for phase in range(2):
            @pl.when(p == phase)
            def _():
                slab, a_v, g_v = slabs[phase], avs[phase], gvs[phase]

                @pl.when(t + 1 < nunits)
                def _():
                    for dsc in in_descs(u + NW, 1 - phase):
                        dsc.start()

                for dsc in in_descs(u, phase):
                    dsc.wait()

                def panel_body(pp, _):
                    for pphase in range(2):
                        @pl.when(pp % 2 == pphase)
                        def _():
                            pan = pans[pphase]

                            @pl.when(t * PPU + pp >= 2)
                            def _():
                                out_desc(u, pp, pan).wait()

                            compute(u >> 2, pp, slab, a_v, g_v, pan)
                            out_desc(u, pp, pan).start()
                    return 0
                lax.fori_loop(0, PPU, panel_body, 0)
        return 0

    lax.fori_loop(0, nunits, unit_body, 0)
    out_desc(0, 0, pans[0]).wait()
    out_desc(0, 0, pans[1]).wait()


def kernel(alpha, mask, basis, missing_basis, alpha_scale, alpha_bias,
           mask_embedding, value_embedding, semantic_matrix, semantic_proj_w,
           categorical_value_mask):
    ve3 = value_embedding.reshape(N, BUCKETS, D)
    catf = categorical_value_mask.astype(jnp.float32)
    aT, gT, t2 = pl.pallas_call(
        _prep_body,
        out_shape=(
            jax.ShapeDtypeStruct((N, B), jnp.float32),
            jax.ShapeDtypeStruct((N, B), jnp.int32),
            jax.ShapeDtypeStruct((N, ROWS, D), jnp.float32),
        ),
    )(alpha.T, mask.T, basis, missing_basis, alpha_scale, alpha_bias,
      mask_embedding, ve3, semantic_matrix, semantic_proj_w, catf)

    mesh = plsc.VectorSubcoreMesh(core_axis_name="c", subcore_axis_name="s")
    sc = pl.kernel(
        _sc_body, mesh=mesh,
        compiler_params=pltpu.CompilerParams(needs_layout_passes=False,
                                             use_tc_tiling_on_sc=False),
        out_type=jax.ShapeDtypeStruct((N, D // 8, B // PAN_B, 8, PAN_B),
                                      jnp.float32),
        scratch_types=[
            pltpu.VMEM((ROWS, D), jnp.float32),
            pltpu.VMEM((ROWS, D), jnp.float32),
            pltpu.VMEM((QB,), jnp.float32),
            pltpu.VMEM((QB,), jnp.float32),
            pltpu.VMEM((QB,), jnp.int32),
            pltpu.VMEM((QB,), jnp.int32),
            pltpu.VMEM((N * D,), jnp.float32),
            pltpu.VMEM((D // 8, 8, PAN_B), jnp.float32),
            pltpu.VMEM((D // 8, 8, PAN_B), jnp.float32),
            pltpu.SemaphoreType.DMA,
            pltpu.SemaphoreType.DMA,
        ],
    )
    out5 = sc(aT, gT, basis.reshape(N * D), t2)
    return out5.transpose(2, 4, 0, 1, 3).reshape(B, N, D)
